# Initial kernel scaffold; baseline (speedup 1.0000x reference)
import jax
import jax.numpy as jnp
from jax.experimental import pallas as pl


def kernel(sentence_map, words, heads_ids, W1, b1, W2, b2, W3, b3, c1w, c1b, c2w, c2b, emb_table):
    raise NotImplementedError("write your pallas kernel here")


# speedup vs baseline: 58.2200x; 58.2200x over previous
import functools

import jax
import jax.numpy as jnp
from jax import lax
from jax.experimental import pallas as pl
from jax.experimental.pallas import tpu as pltpu

GH = 8
WB = 256
CW = 128
ROWS = 144
PAD0 = 8
OFF = 152
ESEG = 432
NEPAD = 8 * ESEG


def _mm_kernel(x_ref, w_ref, b_ref, o_ref):
    o_ref[...] = (
        jnp.dot(x_ref[...], w_ref[...], preferred_element_type=jnp.float32)
        + b_ref[...]
    )


def _emb_kernel(er_ref, w1c_ref, b1_ref, w2_ref, b2_ref, w3_ref, b3_ref,
                ep_ref, z_ref):
    ep_ref[...] = jnp.zeros(ep_ref.shape, jnp.float32)
    e2rev = jnp.dot(er_ref[...], w1c_ref[...],
                    preferred_element_type=jnp.float32)
    for sh in range(8):
        base = sh * ESEG + OFF - sh
        ep_ref[base:base + 128, :] = e2rev
    z1 = jax.nn.relu(b1_ref[...])
    z2 = jax.nn.relu(jnp.dot(z1, w2_ref[...],
                             preferred_element_type=jnp.float32) + b2_ref[...])
    z3 = jnp.dot(z2, w3_ref[...], preferred_element_type=jnp.float32) + b3_ref[...]
    z_ref[...] = jnp.broadcast_to(z3, z_ref.shape)


def _span_kernel(starts_ref, ends_ref, heads_ref, ml_ref,
                 vu_ref, ep_ref, z_ref, w2_ref, b2_ref, w3_ref, b3_ref,
                 c1_ref, c1b_ref, c2_ref, c2b_ref, o0_ref, o1_ref, *, d):
    hb = pl.program_id(0)
    wb = pl.program_id(1)
    o0_ref[...] = jnp.full(o0_ref.shape, -jnp.inf, jnp.float32)
    o1_ref[...] = jnp.full(o1_ref.shape, -jnp.inf, jnp.float32)
    ml = ml_ref[0]
    zrow = z_ref[0:1, :]
    e0 = ep_ref[OFF:OFF + 1, :]
    w2 = w2_ref[...]
    b2 = b2_ref[...]
    w3 = w3_ref[...]
    b3 = b3_ref[...]
    c10 = c1_ref[0:64, :]
    c11 = c1_ref[64:128, :]
    c12 = c1_ref[128:192, :]
    c1b = c1b_ref[...]
    c20 = c2_ref[0:2, :]
    c21 = c2_ref[2:4, :]
    c22 = c2_ref[4:6, :]
    c2b = c2b_ref[...]
    ws0 = wb * WB

    for g in range(GH):
        h = hb * GH + g
        start = starts_ref[h]
        end = ends_ref[h]
        hid = heads_ref[h]
        lenh = end - start
        u_row = vu_ref[pl.ds(hid + PAD0, 1), d:2 * d]
        c_lo = lax.max((start - ws0) // CW, 0)
        c_hi = lax.min((end - ws0 + CW - 1) // CW, WB // CW)

        def c_body(c, inner, g=g, start=start, lenh=lenh, hid=hid,
                   u_row=u_row):
            wc = ws0 + c * CW
            vs = vu_ref[pl.ds(wc, ROWS), 0:d]
            dd = 56 + wc - hid
            sh = (8 - (hid & 7)) & 7
            xa = (lax.min(lax.max(OFF + dd - sh, 0), ESEG - ROWS) // 8) * 8
            es = ep_ref[pl.ds(sh * ESEG + xa, ROWS), :]
            r = lax.broadcasted_iota(jnp.int32, (ROWS, 1), 0)
            q = dd + r
            inr = (q >= 1) & (q <= 127)
            econ = jnp.where(inr, es, e0)
            h1 = jax.nn.relu(vs + econ + u_row)
            h2 = jax.nn.relu(jnp.dot(h1, w2,
                                     preferred_element_type=jnp.float32) + b2)
            h3 = jnp.dot(h2, w3, preferred_element_type=jnp.float32) + b3
            p = (wc - 8 - start) + r
            hext = jnp.where(p < 0, 0.0, jnp.where(p >= lenh, zrow, h3))
            conv1 = (jnp.dot(hext[6:CW + 8], c10,
                             preferred_element_type=jnp.float32)
                     + jnp.dot(hext[7:CW + 9], c11,
                               preferred_element_type=jnp.float32)
                     + jnp.dot(hext[8:CW + 10], c12,
                               preferred_element_type=jnp.float32)
                     + c1b)
            pp = (wc - 1 - start) + lax.broadcasted_iota(
                jnp.int32, (CW + 2, 1), 0)
            c1m = jnp.where((pp >= 0) & (pp < ml), conv1, 0.0)
            dnum = (((1,), (1,)), ((), ()))
            conv2t = (lax.dot_general(c20, c1m[0:CW], dnum,
                                      preferred_element_type=jnp.float32)
                      + lax.dot_general(c21, c1m[1:CW + 1], dnum,
                                        preferred_element_type=jnp.float32)
                      + lax.dot_general(c22, c1m[2:CW + 2], dnum,
                                        preferred_element_type=jnp.float32)
                      + c2b)
            i = lax.broadcasted_iota(jnp.int32, (1, CW), 1)
            j = (wc - start) + i
            valid = (j >= 0) & (j < lenh)
            rel = (hid - wc) - i
            s0 = jnp.where(valid & (rel >= 0), conv2t[0:1, :], -jnp.inf)
            s1 = jnp.where(valid & (rel <= 0), conv2t[1:2, :], -jnp.inf)
            o0_ref[g:g + 1, pl.ds(c * CW, CW)] = s0
            o1_ref[g:g + 1, pl.ds(c * CW, CW)] = s1
            return inner

        lax.fori_loop(c_lo, c_hi, c_body, 0)


def kernel(sentence_map, words, heads_ids, W1, b1, W2, b2, W3, b3,
           c1w, c1b, c2w, c2b, emb_table):
    n_words, d = words.shape
    n_heads = heads_ids.shape[0]
    h2 = W2.shape[1]
    h3 = W3.shape[1]
    npad = n_words + 128

    sh = sentence_map[heads_ids]
    starts = jnp.searchsorted(sentence_map, sh, side='left').astype(jnp.int32)
    ends = jnp.searchsorted(sentence_map, sh, side='right').astype(jnp.int32)
    ml = jnp.max(ends - starts).reshape(1).astype(jnp.int32)
    heads32 = heads_ids.astype(jnp.int32)

    W1a = W1[0:d]
    W1b = W1[d:2 * d]
    W1c = W1[2 * d:]
    wcat = jnp.concatenate([W1b, W1a], axis=1)
    bcat = jnp.concatenate([b1, jnp.zeros((d,), jnp.float32)]).reshape(1, 2 * d)
    words_pad = jnp.pad(words, ((PAD0, 128 - PAD0), (0, 0)))

    rb = npad // 8
    vu = pl.pallas_call(
        _mm_kernel,
        grid=(8,),
        in_specs=[
            pl.BlockSpec((rb, d), lambda i: (i, 0)),
            pl.BlockSpec((d, 2 * d), lambda i: (0, 0)),
            pl.BlockSpec((1, 2 * d), lambda i: (0, 0)),
        ],
        out_specs=pl.BlockSpec((rb, 2 * d), lambda i: (i, 0)),
        out_shape=jax.ShapeDtypeStruct((npad, 2 * d), jnp.float32),
    )(words_pad, wcat, bcat)

    emb_rev = emb_table[::-1]
    b1r = b1.reshape(1, d)
    b2r = b2.reshape(1, h2)
    b3r = b3.reshape(1, h3)
    epad, zrow = pl.pallas_call(
        _emb_kernel,
        out_shape=(jax.ShapeDtypeStruct((NEPAD, d), jnp.float32),
                   jax.ShapeDtypeStruct((8, h3), jnp.float32)),
    )(emb_rev, W1c, b1r, W2, b2r, W3, b3r)

    c1cat = jnp.concatenate([c1w[:, :, k].T for k in range(3)], axis=0)
    c2cat = jnp.concatenate([c2w[:, :, k] for k in range(3)], axis=0)
    c1br = c1b.reshape(1, 4)
    c2br = c2b.reshape(2, 1)

    smem = functools.partial(pl.BlockSpec, memory_space=pltpu.SMEM)
    full = lambda shape: pl.BlockSpec(shape, lambda i, j: tuple(0 for _ in shape))
    o0, o1 = pl.pallas_call(
        functools.partial(_span_kernel, d=d),
        grid=(n_heads // GH, n_words // WB),
        in_specs=[
            smem(), smem(), smem(), smem(),
            full((npad, 2 * d)),
            full((NEPAD, d)),
            full((8, h3)),
            full((d, h2)),
            full((1, h2)),
            full((h2, h3)),
            full((1, h3)),
            full((192, 4)),
            full((1, 4)),
            full((6, 4)),
            full((2, 1)),
        ],
        out_specs=[pl.BlockSpec((GH, WB), lambda i, j: (i, j)),
                   pl.BlockSpec((GH, WB), lambda i, j: (i, j))],
        out_shape=[jax.ShapeDtypeStruct((n_heads, n_words), jnp.float32),
                   jax.ShapeDtypeStruct((n_heads, n_words), jnp.float32)],
    )(starts, ends, heads32, ml, vu, epad, zrow, W2, b2r, W3, b3r,
      c1cat, c1br, c2cat, c2br)

    return jnp.stack([o0, o1], axis=-1)
